# Initial kernel scaffold; baseline (speedup 1.0000x reference)
import jax
import jax.numpy as jnp
from jax.experimental import pallas as pl


def kernel(x, edge_index, W, b):
    raise NotImplementedError("write your pallas kernel here")


# speedup vs baseline: 4.6254x; 4.6254x over previous
import functools

import jax
import jax.numpy as jnp
from jax import lax
from jax.experimental import pallas as pl
from jax.experimental.pallas import tpu as pltpu
from jax.experimental.pallas import tpu_sc as plsc

NC = 2
NS = 16
L = 16


def _rsqrt16(d):
    i = lax.bitcast_convert_type(d, jnp.int32)
    i = jnp.int32(0x5F3759DF) - (i >> 1)
    y = lax.bitcast_convert_type(i, jnp.float32)
    for _ in range(3):
        y = y * (jnp.float32(1.5) - jnp.float32(0.5) * d * y * y)
    return y


def _make_sc_prep(n, npad, e, d):
    per_node = npad // NS
    e_per_tile = e // NS
    ch = 80
    n_chunks = e_per_tile // ch
    assert e_per_tile % ch == 0 and per_node % L == 0

    mesh = plsc.VectorSubcoreMesh(
        core_axis_name="c", subcore_axis_name="s", num_cores=NC,
        num_subcores=NS)

    @functools.partial(
        pl.kernel,
        out_type=[
            jax.ShapeDtypeStruct((n, d), jnp.float32),
            jax.ShapeDtypeStruct((npad,), jnp.float32),
        ],
        mesh=mesh,
        scratch_types=[
            pltpu.VMEM_SHARED((npad,), jnp.float32),
            pltpu.VMEM((ch,), jnp.int32),
            pltpu.VMEM((ch,), jnp.float32),
            pltpu.VMEM((per_node,), jnp.float32),
            pltpu.VMEM((per_node,), jnp.float32),
            pltpu.VMEM((ch, d), jnp.float32),
        ],
        compiler_params=pltpu.CompilerParams(needs_layout_passes=False),
    )
    def sc_prep(x_hbm, src_hbm, dst_hbm, h_hbm, normdst_hbm,
                deg_sh, idx_v, ones_v, deg_l, norm_l, rows_v):
        c = lax.axis_index("c")
        s = lax.axis_index("s")

        def init_body(j, _):
            ones_v[pl.ds(j * L, L)] = jnp.full((L,), 1.0, jnp.float32)
            return 0
        lax.fori_loop(0, ch // L, init_body, 0)

        def zero_body(j, _):
            deg_l[pl.ds(j * L, L)] = jnp.zeros((L,), jnp.float32)
            return 0
        lax.fori_loop(0, per_node // L, zero_body, 0)
        pltpu.sync_copy(deg_l, deg_sh.at[pl.ds(s * per_node, per_node)])
        plsc.subcore_barrier()

        ebase = s * e_per_tile

        def add_chunk(k, _):
            @pl.when(c == 0)
            def _():
                pltpu.sync_copy(src_hbm.at[pl.ds(ebase + k * ch, ch)], idx_v)

            @pl.when(c == 1)
            def _():
                pltpu.sync_copy(dst_hbm.at[pl.ds(ebase + k * ch, ch)], idx_v)

            pltpu.sync_copy(ones_v, deg_sh.at[idx_v], add=True)
            return 0
        lax.fori_loop(0, n_chunks, add_chunk, 0)
        plsc.subcore_barrier()

        nbase = s * per_node
        pltpu.sync_copy(deg_sh.at[pl.ds(nbase, per_node)], deg_l)

        def norm_body(j, _):
            dv = jnp.maximum(deg_l[pl.ds(j * L, L)], jnp.float32(1.0))
            norm_l[pl.ds(j * L, L)] = _rsqrt16(dv)
            return 0
        lax.fori_loop(0, per_node // L, norm_body, 0)

        @pl.when(c == 1)
        def _():
            pltpu.sync_copy(norm_l, normdst_hbm.at[pl.ds(nbase, per_node)])

        @pl.when(c == 0)
        def _():
            nrows = jnp.minimum(per_node, jnp.maximum(n - nbase, 0))
            nch = nrows // ch

            def row_chunk(k, _):
                r0 = nbase + k * ch
                pltpu.sync_copy(x_hbm.at[pl.ds(r0, ch)], rows_v)

                def row_body(r, _):
                    sc = plsc.load_gather(
                        norm_l, [jnp.full((L,), k * ch + r, jnp.int32)])

                    def col_body(j, _):
                        rows_v[r, pl.ds(j * L, L)] = (
                            rows_v[r, pl.ds(j * L, L)] * sc)
                        return 0
                    lax.fori_loop(0, d // L, col_body, 0)
                    return 0
                lax.fori_loop(0, ch, row_body, 0)
                pltpu.sync_copy(rows_v, h_hbm.at[pl.ds(r0, ch)])
                return 0
            lax.fori_loop(0, nch, row_chunk, 0)

    return sc_prep


def _make_sc_agg(n, e, d):
    nw = NC * NS
    e_per_tile = e // nw
    ch = 80
    n_chunks = e_per_tile // ch
    rpt = (n // NS) // 8 * 8
    rpt_last = n - rpt * (NS - 1)
    assert e_per_tile % ch == 0 and rpt % ch in (0, 64) and rpt_last % ch == 0

    mesh = plsc.VectorSubcoreMesh(
        core_axis_name="c", subcore_axis_name="s", num_cores=NC,
        num_subcores=NS)

    @functools.partial(
        pl.kernel,
        out_type=jax.ShapeDtypeStruct((NC, n, d), jnp.float32),
        mesh=mesh,
        scratch_types=[
            pltpu.VMEM_SHARED((n, d), jnp.float32),
            pltpu.VMEM((ch,), jnp.int32),
            pltpu.VMEM((ch,), jnp.int32),
            pltpu.VMEM((ch, d), jnp.float32),
            pltpu.SemaphoreType.DMA,
        ],
        compiler_params=pltpu.CompilerParams(needs_layout_passes=False),
    )
    def sc_agg(h_hbm, src_hbm, dst_hbm, agg_hbm,
               agg_sh, sidx_v, didx_v, rows_v, sem):
        c = lax.axis_index("c")
        s = lax.axis_index("s")

        def zrow(r, _):
            def zcol(j, _):
                rows_v[r, pl.ds(j * L, L)] = jnp.zeros((L,), jnp.float32)
                return 0
            lax.fori_loop(0, d // L, zcol, 0)
            return 0
        lax.fori_loop(0, ch, zrow, 0)

        rbase = s * rpt
        nfull = rpt // ch
        rem = rpt - nfull * ch
        nfull_last = rpt_last // ch

        def zcopy(k, _):
            pltpu.sync_copy(rows_v, agg_sh.at[pl.ds(rbase + k * ch, ch)])
            return 0
        nz = jnp.where(s == NS - 1, nfull_last, nfull)
        lax.fori_loop(0, nz, zcopy, 0)
        if rem:
            @pl.when(s < NS - 1)
            def _():
                pltpu.sync_copy(rows_v.at[pl.ds(0, rem)],
                                agg_sh.at[pl.ds(rbase + nfull * ch, rem)])
        plsc.subcore_barrier()

        ebase = (c * NS + s) * e_per_tile

        def edge_chunk(k, _):
            b = ebase + k * ch
            pltpu.sync_copy(src_hbm.at[pl.ds(b, ch)], sidx_v)
            pltpu.sync_copy(dst_hbm.at[pl.ds(b, ch)], didx_v)
            pltpu.async_copy(h_hbm.at[sidx_v], rows_v, sem).wait()
            pltpu.sync_copy(rows_v, agg_sh.at[didx_v], add=True)
            return 0
        lax.fori_loop(0, n_chunks, edge_chunk, 0)
        plsc.subcore_barrier()

        @pl.when(s < NS - 1)
        def _():
            pltpu.sync_copy(agg_sh.at[pl.ds(rbase, rpt)],
                            agg_hbm.at[c, pl.ds(rbase, rpt)])

        @pl.when(s == NS - 1)
        def _():
            pltpu.sync_copy(agg_sh.at[pl.ds(rbase, rpt_last)],
                            agg_hbm.at[c, pl.ds(rbase, rpt_last)])

    return sc_agg


def _tc_final_body(agg_ref, nd_ref, w_ref, b_ref, o_ref):
    a = (agg_ref[0] + agg_ref[1]) * nd_ref[...]
    w = jnp.maximum(w_ref[...], 0.0)
    r = jnp.dot(a, w, preferred_element_type=jnp.float32) + b_ref[...]
    o_ref[...] = jnp.where(r >= 0, r, jnp.float32(0.01) * r)


def _make_tc_final(n, d, blk):
    grid = n // blk
    return pl.pallas_call(
        _tc_final_body,
        grid=(grid,),
        in_specs=[
            pl.BlockSpec((NC, blk, d), lambda i: (0, i, 0)),
            pl.BlockSpec((blk, 1), lambda i: (i, 0)),
            pl.BlockSpec((d, d), lambda i: (0, 0)),
            pl.BlockSpec((1, d), lambda i: (0, 0)),
        ],
        out_specs=pl.BlockSpec((blk, d), lambda i: (i, 0)),
        out_shape=jax.ShapeDtypeStruct((n, d), jnp.float32),
    )


@jax.jit
def kernel(x, edge_index, W, b):
    n, d = x.shape
    e = edge_index.shape[1]
    npad = ((n + NC * NS * L - 1) // (NC * NS * L)) * NC * NS * L

    src = edge_index[0]
    dst = edge_index[1]

    h, norm_dst = _make_sc_prep(n, npad, e, d)(x, src, dst)
    aggp = _make_sc_agg(n, e, d)(h, src, dst)
    out = _make_tc_final(n, d, 1000)(
        aggp, norm_dst[:n, None], W, b[None, :])
    return out


# speedup vs baseline: 8.7105x; 1.8832x over previous
import functools

import jax
import jax.numpy as jnp
from jax import lax
from jax.experimental import pallas as pl
from jax.experimental.pallas import tpu as pltpu
from jax.experimental.pallas import tpu_sc as plsc

NC = 2
NS = 16
L = 16
CH = 80


def _rsqrt16(d):
    i = lax.bitcast_convert_type(d, jnp.int32)
    i = jnp.int32(0x5F3759DF) - (i >> 1)
    y = lax.bitcast_convert_type(i, jnp.float32)
    for _ in range(3):
        y = y * (jnp.float32(1.5) - jnp.float32(0.5) * d * y * y)
    return y


def _zero_vec(ref, n):
    def body(j, _):
        ref[pl.ds(j * L, L)] = jnp.zeros((L,), jnp.float32)
        return 0
    lax.fori_loop(0, n // L, body, 0)


def _zero_rows(ref, rows, cols):
    def rbody(r, _):
        def cbody(j, _):
            ref[r, pl.ds(j * L, L)] = jnp.zeros((L,), jnp.float32)
            return 0
        lax.fori_loop(0, cols // L, cbody, 0)
        return 0
    lax.fori_loop(0, rows, rbody, 0)


def _make_sc_prep(n, npad, e, d):
    per_node = npad // NS
    e_per_tile = e // NS
    nch = e_per_tile // CH
    rch = 80
    assert e_per_tile % CH == 0 and per_node % rch == 0 and n % rch == 0

    mesh = plsc.VectorSubcoreMesh(
        core_axis_name="c", subcore_axis_name="s", num_cores=NC,
        num_subcores=NS)

    @functools.partial(
        pl.kernel,
        out_type=[
            jax.ShapeDtypeStruct((n, d), jnp.float32),
            jax.ShapeDtypeStruct((npad,), jnp.float32),
        ],
        mesh=mesh,
        scratch_types=[
            pltpu.VMEM_SHARED((npad,), jnp.float32),
            pltpu.VMEM((CH,), jnp.int32),
            pltpu.VMEM((CH,), jnp.int32),
            pltpu.VMEM((CH,), jnp.float32),
            pltpu.VMEM((per_node,), jnp.float32),
            pltpu.VMEM((per_node,), jnp.float32),
            pltpu.VMEM((rch, d), jnp.float32),
            pltpu.VMEM((rch, d), jnp.float32),
            pltpu.SemaphoreType.DMA,
            pltpu.SemaphoreType.DMA,
            pltpu.SemaphoreType.DMA,
            pltpu.SemaphoreType.DMA,
            pltpu.SemaphoreType.DMA,
            pltpu.SemaphoreType.DMA,
            pltpu.SemaphoreType.DMA,
            pltpu.SemaphoreType.DMA,
        ],
        compiler_params=pltpu.CompilerParams(needs_layout_passes=False),
    )
    def sc_prep(x_hbm, src_hbm, dst_hbm, h_hbm, normdst_hbm,
                deg_sh, ia, ib, ones_v, deg_l, norm_l, rows_a, rows_b,
                lia, lib_, osa, osb, lsa, lsb, ssa, ssb):
        c = lax.axis_index("c")
        s = lax.axis_index("s")
        ebase = s * e_per_tile
        idxb = (ia, ib)
        lsem = (lia, lib_)
        osem = (osa, osb)

        def i_start(k, p):
            @pl.when(c == 0)
            def _():
                pltpu.async_copy(
                    src_hbm.at[pl.ds(ebase + k * CH, CH)], idxb[p], lsem[p])

            @pl.when(c == 1)
            def _():
                pltpu.async_copy(
                    dst_hbm.at[pl.ds(ebase + k * CH, CH)], idxb[p], lsem[p])

        def i_wait(p):
            pltpu.make_async_copy(
                src_hbm.at[pl.ds(0, CH)], idxb[p], lsem[p]).wait()

        def a_start(k, p):
            pltpu.async_copy(ones_v, deg_sh.at[idxb[p]], osem[p], add=True)

        def a_wait(p):
            pltpu.make_async_copy(ones_v, deg_sh.at[idxb[p]], osem[p]).wait()

        i_start(0, 0)

        def ones_body(j, _):
            ones_v[pl.ds(j * L, L)] = jnp.full((L,), 1.0, jnp.float32)
            return 0
        lax.fori_loop(0, CH // L, ones_body, 0)
        _zero_vec(deg_l, per_node)
        pltpu.sync_copy(deg_l, deg_sh.at[pl.ds(s * per_node, per_node)])
        plsc.subcore_barrier()

        def dstep(k, p):
            po = 1 - p

            @pl.when(k + 1 < nch)
            def _():
                @pl.when(k >= 1)
                def _():
                    a_wait(po)
                i_start(k + 1, po)
            i_wait(p)
            a_start(k, p)

        def dpipe(kk, _):
            dstep(kk * 2, 0)
            dstep(kk * 2 + 1, 1)
            return 0
        lax.fori_loop(0, nch // 2, dpipe, 0)
        for t in range(nch - nch % 2, nch):
            dstep(t, t % 2)
        a_wait((nch - 2) % 2)
        a_wait((nch - 1) % 2)
        plsc.subcore_barrier()

        nbase = s * per_node
        pltpu.sync_copy(deg_sh.at[pl.ds(nbase, per_node)], deg_l)

        def norm_body(j, _):
            dv = jnp.maximum(deg_l[pl.ds(j * L, L)], jnp.float32(1.0))
            norm_l[pl.ds(j * L, L)] = _rsqrt16(dv)
            return 0
        lax.fori_loop(0, per_node // L, norm_body, 0)

        @pl.when(c == 1)
        def _():
            pltpu.sync_copy(norm_l, normdst_hbm.at[pl.ds(nbase, per_node)])

        @pl.when(c == 0)
        def _():
            nrows = jnp.minimum(per_node, jnp.maximum(n - nbase, 0))
            nrch = nrows // rch

            def load(q, buf, sem):
                pltpu.async_copy(
                    x_hbm.at[pl.ds(nbase + q * rch, rch)], buf, sem)

            def wait_load(buf, sem):
                pltpu.make_async_copy(x_hbm.at[pl.ds(0, rch)], buf, sem).wait()

            def store(q, buf, sem):
                pltpu.async_copy(
                    buf, h_hbm.at[pl.ds(nbase + q * rch, rch)], sem)

            def wait_store(buf, sem):
                pltpu.make_async_copy(buf, h_hbm.at[pl.ds(0, rch)], sem).wait()

            def scale(q, buf):
                def row_body(r, _):
                    sc = plsc.load_gather(
                        norm_l, [jnp.full((L,), q * rch + r, jnp.int32)])

                    def col_body(j, _):
                        buf[r, pl.ds(j * L, L)] = buf[r, pl.ds(j * L, L)] * sc
                        return 0
                    lax.fori_loop(0, d // L, col_body, 0)
                    return 0
                lax.fori_loop(0, rch, row_body, 0)

            load(0, rows_a, lsa)

            def hpipe(qq, _):
                q = qq * 2
                @pl.when(q + 1 < nrch)
                def _():
                    @pl.when(q >= 1)
                    def _():
                        wait_store(rows_b, ssb)
                    load(q + 1, rows_b, lsb)
                wait_load(rows_a, lsa)
                scale(q, rows_a)
                store(q, rows_a, ssa)
                @pl.when(q + 1 < nrch)
                def _():
                    @pl.when(q + 2 < nrch)
                    def _():
                        wait_store(rows_a, ssa)
                        load(q + 2, rows_a, lsa)
                    wait_load(rows_b, lsb)
                    scale(q + 1, rows_b)
                    store(q + 1, rows_b, ssb)
                return 0
            lax.fori_loop(0, (nrch + 1) // 2, hpipe, 0)
            @pl.when(nrch >= 2)
            def _():
                wait_store(rows_b, ssb)
            wait_store(rows_a, ssa)

    return sc_prep


def _make_sc_agg(n, e, d):
    nw = NC * NS
    e_per_tile = e // nw
    nch = e_per_tile // CH
    rpt = (n // NS) // 8 * 8
    rpt_last = n - rpt * (NS - 1)
    assert e_per_tile % CH == 0 and nch >= 4

    mesh = plsc.VectorSubcoreMesh(
        core_axis_name="c", subcore_axis_name="s", num_cores=NC,
        num_subcores=NS)

    @functools.partial(
        pl.kernel,
        out_type=jax.ShapeDtypeStruct((NC, n, d), jnp.float32),
        mesh=mesh,
        scratch_types=[
            pltpu.VMEM_SHARED((n, d), jnp.float32),
            pltpu.VMEM((CH,), jnp.int32),
            pltpu.VMEM((CH,), jnp.int32),
            pltpu.VMEM((CH,), jnp.int32),
            pltpu.VMEM((CH,), jnp.int32),
            pltpu.VMEM((CH, d), jnp.float32),
            pltpu.VMEM((CH, d), jnp.float32),
            pltpu.SemaphoreType.DMA,
            pltpu.SemaphoreType.DMA,
            pltpu.SemaphoreType.DMA,
            pltpu.SemaphoreType.DMA,
            pltpu.SemaphoreType.DMA,
            pltpu.SemaphoreType.DMA,
            pltpu.SemaphoreType.DMA,
            pltpu.SemaphoreType.DMA,
        ],
        compiler_params=pltpu.CompilerParams(needs_layout_passes=False),
    )
    def sc_agg(h_hbm, src_hbm, dst_hbm, agg_hbm,
               agg_sh, sia, sib, dia, dib, ba, bb,
               sla, slb, dla, dlb, ga, gb, sa, sb):
        c = lax.axis_index("c")
        s = lax.axis_index("s")
        wid = c * NS + s
        ebase = wid * e_per_tile
        sidx = (sia, sib)
        didx = (dia, dib)
        bufs = (ba, bb)
        slsem = (sla, slb)
        dlsem = (dla, dlb)
        gsem = (ga, gb)
        ssem = (sa, sb)

        def i_start(k, p):
            pltpu.async_copy(
                src_hbm.at[pl.ds(ebase + k * CH, CH)], sidx[p], slsem[p])
            pltpu.async_copy(
                dst_hbm.at[pl.ds(ebase + k * CH, CH)], didx[p], dlsem[p])

        def i_wait(p):
            pltpu.make_async_copy(
                src_hbm.at[pl.ds(0, CH)], sidx[p], slsem[p]).wait()
            pltpu.make_async_copy(
                src_hbm.at[pl.ds(0, CH)], didx[p], dlsem[p]).wait()

        def g_start(p):
            pltpu.async_copy(h_hbm.at[sidx[p]], bufs[p], gsem[p])

        def g_wait(p):
            pltpu.make_async_copy(h_hbm.at[sidx[p]], bufs[p], gsem[p]).wait()

        def s_start(p):
            pltpu.async_copy(bufs[p], agg_sh.at[didx[p]], ssem[p], add=True)

        def s_wait(p):
            pltpu.make_async_copy(
                bufs[p], agg_sh.at[didx[p]], ssem[p]).wait()

        i_start(0, 0)
        _zero_rows(ba, CH, d)
        rbase = s * rpt

        @pl.when(s < NS - 1)
        def _():
            def zc(k, _):
                pltpu.sync_copy(ba, agg_sh.at[pl.ds(rbase + k * CH, CH)])
                return 0
            lax.fori_loop(0, rpt // CH, zc, 0)
            rem = rpt - (rpt // CH) * CH
            if rem:
                pltpu.sync_copy(
                    ba.at[pl.ds(0, rem)],
                    agg_sh.at[pl.ds(rbase + (rpt // CH) * CH, rem)])

        @pl.when(s == NS - 1)
        def _():
            def zc(k, _):
                pltpu.sync_copy(ba, agg_sh.at[pl.ds(rbase + k * CH, CH)])
                return 0
            lax.fori_loop(0, rpt_last // CH, zc, 0)
            rem = rpt_last - (rpt_last // CH) * CH
            if rem:
                pltpu.sync_copy(
                    ba.at[pl.ds(0, rem)],
                    agg_sh.at[pl.ds(rbase + (rpt_last // CH) * CH, rem)])

        plsc.subcore_barrier()

        i_wait(0)
        g_start(0)

        def step1(k, p):
            po = 1 - p

            @pl.when(k + 1 < nch)
            def _():
                @pl.when(k >= 1)
                def _():
                    s_wait(po)
                i_start(k + 1, po)
                i_wait(po)
                g_start(po)
            g_wait(p)
            s_start(p)

        def pipe1(kk, _):
            step1(kk * 2, 0)
            step1(kk * 2 + 1, 1)
            return 0
        lax.fori_loop(0, nch // 2, pipe1, 0)
        for t in range(nch - nch % 2, nch):
            step1(t, t % 2)
        s_wait((nch - 2) % 2)
        s_wait((nch - 1) % 2)
        plsc.subcore_barrier()

        @pl.when(s < NS - 1)
        def _():
            pltpu.sync_copy(agg_sh.at[pl.ds(rbase, rpt)],
                            agg_hbm.at[c, pl.ds(rbase, rpt)])

        @pl.when(s == NS - 1)
        def _():
            pltpu.sync_copy(agg_sh.at[pl.ds(rbase, rpt_last)],
                            agg_hbm.at[c, pl.ds(rbase, rpt_last)])

    return sc_agg


def _tc_final_body(agg_ref, nd_ref, w_ref, b_ref, o_ref):
    a = (agg_ref[0] + agg_ref[1]) * nd_ref[...]
    w = jnp.maximum(w_ref[...], 0.0)
    r = jnp.dot(a, w, preferred_element_type=jnp.float32) + b_ref[...]
    o_ref[...] = jnp.where(r >= 0, r, jnp.float32(0.01) * r)


def _make_tc_final(n, d, blk):
    grid = n // blk
    return pl.pallas_call(
        _tc_final_body,
        grid=(grid,),
        in_specs=[
            pl.BlockSpec((NC, blk, d), lambda i: (0, i, 0)),
            pl.BlockSpec((blk, 1), lambda i: (i, 0)),
            pl.BlockSpec((d, d), lambda i: (0, 0)),
            pl.BlockSpec((1, d), lambda i: (0, 0)),
        ],
        out_specs=pl.BlockSpec((blk, d), lambda i: (i, 0)),
        out_shape=jax.ShapeDtypeStruct((n, d), jnp.float32),
    )


@jax.jit
def kernel(x, edge_index, W, b):
    n, d = x.shape
    e = edge_index.shape[1]
    npad = ((n + NC * NS * L - 1) // (NC * NS * L)) * NC * NS * L

    src = edge_index[0]
    dst = edge_index[1]

    h, norm_dst = _make_sc_prep(n, npad, e, d)(x, src, dst)
    aggp = _make_sc_agg(n, e, d)(h, src, dst)
    out = _make_tc_final(n, d, 1000)(
        aggp, norm_dst[:n, None], W, b[None, :])
    return out


# speedup vs baseline: 10.3506x; 1.1883x over previous
import functools

import jax
import jax.numpy as jnp
from jax import lax
from jax.experimental import pallas as pl
from jax.experimental.pallas import tpu as pltpu
from jax.experimental.pallas import tpu_sc as plsc

NC = 2
NS = 16
L = 16
CH = 80


def _rsqrt16(d):
    i = lax.bitcast_convert_type(d, jnp.int32)
    i = jnp.int32(0x5F3759DF) - (i >> 1)
    y = lax.bitcast_convert_type(i, jnp.float32)
    for _ in range(3):
        y = y * (jnp.float32(1.5) - jnp.float32(0.5) * d * y * y)
    return y


def _zero_vec(ref, n):
    def body(j, _):
        ref[pl.ds(j * L, L)] = jnp.zeros((L,), jnp.float32)
        return 0
    lax.fori_loop(0, n // L, body, 0)


def _zero_rows(ref, rows, cols):
    def rbody(r, _):
        def cbody(j, _):
            ref[r, pl.ds(j * L, L)] = jnp.zeros((L,), jnp.float32)
            return 0
        lax.fori_loop(0, cols // L, cbody, 0)
        return 0
    lax.fori_loop(0, rows, rbody, 0)


def _make_sc_prep(n, npad, e, d):
    per_node = npad // NS
    e_per_tile = e // NS
    nch = e_per_tile // CH
    rch = 80
    assert e_per_tile % CH == 0 and per_node % rch == 0 and n % rch == 0

    mesh = plsc.VectorSubcoreMesh(
        core_axis_name="c", subcore_axis_name="s", num_cores=NC,
        num_subcores=NS)

    @functools.partial(
        pl.kernel,
        out_type=[
            jax.ShapeDtypeStruct((n, d), jnp.float32),
            jax.ShapeDtypeStruct((npad,), jnp.float32),
        ],
        mesh=mesh,
        scratch_types=[
            pltpu.VMEM_SHARED((npad,), jnp.float32),
            pltpu.VMEM((CH,), jnp.int32),
            pltpu.VMEM((CH,), jnp.int32),
            pltpu.VMEM((CH,), jnp.int32),
            pltpu.VMEM((CH,), jnp.int32),
            pltpu.VMEM((CH,), jnp.float32),
            pltpu.VMEM((per_node,), jnp.float32),
            pltpu.VMEM((per_node,), jnp.float32),
            pltpu.VMEM((rch, d), jnp.float32),
            pltpu.VMEM((rch, d), jnp.float32),
            pltpu.SemaphoreType.DMA,
            pltpu.SemaphoreType.DMA,
            pltpu.SemaphoreType.DMA,
            pltpu.SemaphoreType.DMA,
            pltpu.SemaphoreType.DMA,
            pltpu.SemaphoreType.DMA,
            pltpu.SemaphoreType.DMA,
            pltpu.SemaphoreType.DMA,
            pltpu.SemaphoreType.DMA,
            pltpu.SemaphoreType.DMA,
            pltpu.SemaphoreType.DMA,
            pltpu.SemaphoreType.DMA,
        ],
        compiler_params=pltpu.CompilerParams(needs_layout_passes=False),
    )
    def sc_prep(x_hbm, src_hbm, dst_hbm, h_hbm, normdst_hbm,
                deg_sh, i0, i1, i2, i3, ones_v, deg_l, norm_l,
                rows_a, rows_b,
                li0, li1, li2, li3, os0, os1, os2, os3,
                lsa, lsb, ssa, ssb):
        c = lax.axis_index("c")
        s = lax.axis_index("s")
        ebase = s * e_per_tile
        idxb = (i0, i1, i2, i3)
        lsem = (li0, li1, li2, li3)
        osem = (os0, os1, os2, os3)

        def i_start(k, p):
            @pl.when(c == 0)
            def _():
                pltpu.async_copy(
                    src_hbm.at[pl.ds(ebase + k * CH, CH)], idxb[p], lsem[p])

            @pl.when(c == 1)
            def _():
                pltpu.async_copy(
                    dst_hbm.at[pl.ds(ebase + k * CH, CH)], idxb[p], lsem[p])

        def i_wait(p):
            pltpu.make_async_copy(
                src_hbm.at[pl.ds(0, CH)], idxb[p], lsem[p]).wait()

        def a_start(k, p):
            pltpu.async_copy(ones_v, deg_sh.at[idxb[p]], osem[p], add=True)

        def a_wait(p):
            pltpu.make_async_copy(ones_v, deg_sh.at[idxb[p]], osem[p]).wait()

        for j in range(3):
            i_start(j, j)

        def ones_body(j, _):
            ones_v[pl.ds(j * L, L)] = jnp.full((L,), 1.0, jnp.float32)
            return 0
        lax.fori_loop(0, CH // L, ones_body, 0)
        _zero_vec(deg_l, per_node)
        pltpu.sync_copy(deg_l, deg_sh.at[pl.ds(s * per_node, per_node)])
        plsc.subcore_barrier()

        def dstep(k, p):
            p3 = (p + 3) % 4

            @pl.when(k + 3 < nch)
            def _():
                @pl.when(k >= 1)
                def _():
                    a_wait(p3)
                i_start(k + 3, p3)
            i_wait(p)
            a_start(k, p)

        def dpipe(kk, _):
            for b in range(4):
                dstep(kk * 4 + b, b)
            return 0
        lax.fori_loop(0, nch // 4, dpipe, 0)
        for t in range(nch - nch % 4, nch):
            dstep(t, t % 4)
        for t in range(nch - 4, nch):
            a_wait(t % 4)
        plsc.subcore_barrier()

        nbase = s * per_node
        pltpu.sync_copy(deg_sh.at[pl.ds(nbase, per_node)], deg_l)

        def norm_body(j, _):
            dv = jnp.maximum(deg_l[pl.ds(j * L, L)], jnp.float32(1.0))
            norm_l[pl.ds(j * L, L)] = _rsqrt16(dv)
            return 0
        lax.fori_loop(0, per_node // L, norm_body, 0)

        @pl.when(c == 1)
        def _():
            pltpu.sync_copy(norm_l, normdst_hbm.at[pl.ds(nbase, per_node)])

        @pl.when(c == 0)
        def _():
            nrows = jnp.minimum(per_node, jnp.maximum(n - nbase, 0))
            nrch = nrows // rch

            def load(q, buf, sem):
                pltpu.async_copy(
                    x_hbm.at[pl.ds(nbase + q * rch, rch)], buf, sem)

            def wait_load(buf, sem):
                pltpu.make_async_copy(x_hbm.at[pl.ds(0, rch)], buf, sem).wait()

            def store(q, buf, sem):
                pltpu.async_copy(
                    buf, h_hbm.at[pl.ds(nbase + q * rch, rch)], sem)

            def wait_store(buf, sem):
                pltpu.make_async_copy(buf, h_hbm.at[pl.ds(0, rch)], sem).wait()

            def scale(q, buf):
                def row_body(r, _):
                    sc = plsc.load_gather(
                        norm_l, [jnp.full((L,), q * rch + r, jnp.int32)])

                    def col_body(j, _):
                        buf[r, pl.ds(j * L, L)] = buf[r, pl.ds(j * L, L)] * sc
                        return 0
                    lax.fori_loop(0, d // L, col_body, 0)
                    return 0
                lax.fori_loop(0, rch, row_body, 0)

            load(0, rows_a, lsa)

            def hpipe(qq, _):
                q = qq * 2
                @pl.when(q + 1 < nrch)
                def _():
                    @pl.when(q >= 1)
                    def _():
                        wait_store(rows_b, ssb)
                    load(q + 1, rows_b, lsb)
                wait_load(rows_a, lsa)
                scale(q, rows_a)
                store(q, rows_a, ssa)
                @pl.when(q + 1 < nrch)
                def _():
                    @pl.when(q + 2 < nrch)
                    def _():
                        wait_store(rows_a, ssa)
                        load(q + 2, rows_a, lsa)
                    wait_load(rows_b, lsb)
                    scale(q + 1, rows_b)
                    store(q + 1, rows_b, ssb)
                return 0
            lax.fori_loop(0, (nrch + 1) // 2, hpipe, 0)
            @pl.when(nrch >= 2)
            def _():
                wait_store(rows_b, ssb)
            wait_store(rows_a, ssa)

    return sc_prep


def _make_sc_agg(n, e, d):
    nw = NC * NS
    e_per_tile = e // nw
    nch = e_per_tile // CH
    rpt = (n // NS) // 8 * 8
    rpt_last = n - rpt * (NS - 1)
    assert e_per_tile % CH == 0 and nch >= 4

    mesh = plsc.VectorSubcoreMesh(
        core_axis_name="c", subcore_axis_name="s", num_cores=NC,
        num_subcores=NS)

    @functools.partial(
        pl.kernel,
        out_type=jax.ShapeDtypeStruct((NC, n, d), jnp.float32),
        mesh=mesh,
        scratch_types=[
            pltpu.VMEM_SHARED((n, d), jnp.float32),
            pltpu.VMEM((CH,), jnp.int32),
            pltpu.VMEM((CH,), jnp.int32),
            pltpu.VMEM((CH,), jnp.int32),
            pltpu.VMEM((CH,), jnp.int32),
            pltpu.VMEM((CH,), jnp.int32),
            pltpu.VMEM((CH,), jnp.int32),
            pltpu.VMEM((CH,), jnp.int32),
            pltpu.VMEM((CH,), jnp.int32),
            pltpu.VMEM((CH, d), jnp.float32),
            pltpu.VMEM((CH, d), jnp.float32),
            pltpu.VMEM((CH, d), jnp.float32),
            pltpu.VMEM((CH, d), jnp.float32),
            pltpu.SemaphoreType.DMA,
            pltpu.SemaphoreType.DMA,
            pltpu.SemaphoreType.DMA,
            pltpu.SemaphoreType.DMA,
            pltpu.SemaphoreType.DMA,
            pltpu.SemaphoreType.DMA,
            pltpu.SemaphoreType.DMA,
            pltpu.SemaphoreType.DMA,
            pltpu.SemaphoreType.DMA,
            pltpu.SemaphoreType.DMA,
            pltpu.SemaphoreType.DMA,
            pltpu.SemaphoreType.DMA,
            pltpu.SemaphoreType.DMA,
            pltpu.SemaphoreType.DMA,
            pltpu.SemaphoreType.DMA,
            pltpu.SemaphoreType.DMA,
        ],
        compiler_params=pltpu.CompilerParams(needs_layout_passes=False),
    )
    def sc_agg(h_hbm, src_hbm, dst_hbm, agg_hbm,
               agg_sh, si0, si1, si2, si3, di0, di1, di2, di3,
               rb0, rb1, rb2, rb3,
               sl0, sl1, sl2, sl3, dl0, dl1, dl2, dl3,
               g0, g1, g2, g3, ss0, ss1, ss2, ss3):
        c = lax.axis_index("c")
        s = lax.axis_index("s")
        wid = c * NS + s
        ebase = wid * e_per_tile
        sidx = (si0, si1, si2, si3)
        didx = (di0, di1, di2, di3)
        bufs = (rb0, rb1, rb2, rb3)
        slsem = (sl0, sl1, sl2, sl3)
        dlsem = (dl0, dl1, dl2, dl3)
        gsem = (g0, g1, g2, g3)
        ssem = (ss0, ss1, ss2, ss3)

        def i_start(k, p):
            pltpu.async_copy(
                src_hbm.at[pl.ds(ebase + k * CH, CH)], sidx[p], slsem[p])
            pltpu.async_copy(
                dst_hbm.at[pl.ds(ebase + k * CH, CH)], didx[p], dlsem[p])

        def i_wait(p):
            pltpu.make_async_copy(
                src_hbm.at[pl.ds(0, CH)], sidx[p], slsem[p]).wait()
            pltpu.make_async_copy(
                src_hbm.at[pl.ds(0, CH)], didx[p], dlsem[p]).wait()

        def g_start(p):
            pltpu.async_copy(h_hbm.at[sidx[p]], bufs[p], gsem[p])

        def g_wait(p):
            pltpu.make_async_copy(h_hbm.at[sidx[p]], bufs[p], gsem[p]).wait()

        def s_start(p):
            pltpu.async_copy(bufs[p], agg_sh.at[didx[p]], ssem[p], add=True)

        def s_wait(p):
            pltpu.make_async_copy(
                bufs[p], agg_sh.at[didx[p]], ssem[p]).wait()

        for j in range(3):
            i_start(j, j)
        ba = rb0
        _zero_rows(ba, CH, d)
        rbase = s * rpt

        @pl.when(s < NS - 1)
        def _():
            def zc(k, _):
                pltpu.sync_copy(ba, agg_sh.at[pl.ds(rbase + k * CH, CH)])
                return 0
            lax.fori_loop(0, rpt // CH, zc, 0)
            rem = rpt - (rpt // CH) * CH
            if rem:
                pltpu.sync_copy(
                    ba.at[pl.ds(0, rem)],
                    agg_sh.at[pl.ds(rbase + (rpt // CH) * CH, rem)])

        @pl.when(s == NS - 1)
        def _():
            def zc(k, _):
                pltpu.sync_copy(ba, agg_sh.at[pl.ds(rbase + k * CH, CH)])
                return 0
            lax.fori_loop(0, rpt_last // CH, zc, 0)
            rem = rpt_last - (rpt_last // CH) * CH
            if rem:
                pltpu.sync_copy(
                    ba.at[pl.ds(0, rem)],
                    agg_sh.at[pl.ds(rbase + (rpt_last // CH) * CH, rem)])

        plsc.subcore_barrier()

        for j in range(3):
            i_wait(j)
            g_start(j)

        def step1(k, p):
            p3 = (p + 3) % 4

            @pl.when(k + 3 < nch)
            def _():
                @pl.when(k >= 1)
                def _():
                    s_wait(p3)
                i_start(k + 3, p3)
                i_wait(p3)
                g_start(p3)
            g_wait(p)
            s_start(p)

        def pipe1(kk, _):
            for b in range(4):
                step1(kk * 4 + b, b)
            return 0
        lax.fori_loop(0, nch // 4, pipe1, 0)
        for t in range(nch - nch % 4, nch):
            step1(t, t % 4)
        for t in range(nch - 4, nch):
            s_wait(t % 4)
        plsc.subcore_barrier()

        @pl.when(s < NS - 1)
        def _():
            pltpu.sync_copy(agg_sh.at[pl.ds(rbase, rpt)],
                            agg_hbm.at[c, pl.ds(rbase, rpt)])

        @pl.when(s == NS - 1)
        def _():
            pltpu.sync_copy(agg_sh.at[pl.ds(rbase, rpt_last)],
                            agg_hbm.at[c, pl.ds(rbase, rpt_last)])

    return sc_agg


def _tc_final_body(agg_ref, nd_ref, w_ref, b_ref, o_ref):
    a = (agg_ref[0] + agg_ref[1]) * nd_ref[...]
    w = jnp.maximum(w_ref[...], 0.0)
    r = jnp.dot(a, w, preferred_element_type=jnp.float32) + b_ref[...]
    o_ref[...] = jnp.where(r >= 0, r, jnp.float32(0.01) * r)


def _make_tc_final(n, d, blk):
    grid = n // blk
    return pl.pallas_call(
        _tc_final_body,
        grid=(grid,),
        in_specs=[
            pl.BlockSpec((NC, blk, d), lambda i: (0, i, 0)),
            pl.BlockSpec((blk, 1), lambda i: (i, 0)),
            pl.BlockSpec((d, d), lambda i: (0, 0)),
            pl.BlockSpec((1, d), lambda i: (0, 0)),
        ],
        out_specs=pl.BlockSpec((blk, d), lambda i: (i, 0)),
        out_shape=jax.ShapeDtypeStruct((n, d), jnp.float32),
    )


@jax.jit
def kernel(x, edge_index, W, b):
    n, d = x.shape
    e = edge_index.shape[1]
    npad = ((n + NC * NS * L - 1) // (NC * NS * L)) * NC * NS * L

    src = edge_index[0]
    dst = edge_index[1]

    h, norm_dst = _make_sc_prep(n, npad, e, d)(x, src, dst)
    aggp = _make_sc_agg(n, e, d)(h, src, dst)
    out = _make_tc_final(n, d, 1000)(
        aggp, norm_dst[:n, None], W, b[None, :])
    return out


# speedup vs baseline: 12.6936x; 1.2264x over previous
import functools

import jax
import jax.numpy as jnp
from jax import lax
from jax.experimental import pallas as pl
from jax.experimental.pallas import tpu as pltpu
from jax.experimental.pallas import tpu_sc as plsc

NC = 2
NS = 16
L = 16
CH = 80


def _rsqrt16(d):
    i = lax.bitcast_convert_type(d, jnp.int32)
    i = jnp.int32(0x5F3759DF) - (i >> 1)
    y = lax.bitcast_convert_type(i, jnp.float32)
    for _ in range(3):
        y = y * (jnp.float32(1.5) - jnp.float32(0.5) * d * y * y)
    return y


def _zero_vec(ref, n):
    def body(j, _):
        ref[pl.ds(j * L, L)] = jnp.zeros((L,), jnp.float32)
        return 0
    lax.fori_loop(0, n // L, body, 0)


def _zero_rows(ref, rows, cols):
    def rbody(r, _):
        def cbody(j, _):
            ref[r, pl.ds(j * L, L)] = jnp.zeros((L,), jnp.float32)
            return 0
        lax.fori_loop(0, cols // L, cbody, 0)
        return 0
    lax.fori_loop(0, rows, rbody, 0)


def _make_sc_prep(n, npad, e, d):
    per_node = npad // NS
    e_per_tile = e // NS
    nch = e_per_tile // CH
    rch = 80
    assert e_per_tile % CH == 0 and per_node % rch == 0 and n % rch == 0

    mesh = plsc.VectorSubcoreMesh(
        core_axis_name="c", subcore_axis_name="s", num_cores=NC,
        num_subcores=NS)

    @functools.partial(
        pl.kernel,
        out_type=[
            jax.ShapeDtypeStruct((n, d), jnp.float32),
            jax.ShapeDtypeStruct((npad,), jnp.float32),
        ],
        mesh=mesh,
        scratch_types=[
            pltpu.VMEM_SHARED((npad,), jnp.float32),
            pltpu.VMEM((CH,), jnp.int32),
            pltpu.VMEM((CH,), jnp.int32),
            pltpu.VMEM((CH,), jnp.int32),
            pltpu.VMEM((CH,), jnp.int32),
            pltpu.VMEM((CH,), jnp.float32),
            pltpu.VMEM((per_node,), jnp.float32),
            pltpu.VMEM((per_node,), jnp.float32),
            pltpu.VMEM((rch, d), jnp.float32),
            pltpu.VMEM((rch, d), jnp.float32),
            pltpu.SemaphoreType.DMA,
            pltpu.SemaphoreType.DMA,
            pltpu.SemaphoreType.DMA,
            pltpu.SemaphoreType.DMA,
            pltpu.SemaphoreType.DMA,
            pltpu.SemaphoreType.DMA,
            pltpu.SemaphoreType.DMA,
            pltpu.SemaphoreType.DMA,
            pltpu.SemaphoreType.DMA,
            pltpu.SemaphoreType.DMA,
            pltpu.SemaphoreType.DMA,
            pltpu.SemaphoreType.DMA,
        ],
        compiler_params=pltpu.CompilerParams(needs_layout_passes=False),
    )
    def sc_prep(x_hbm, src_hbm, dst_hbm, h_hbm, normdst_hbm,
                deg_sh, i0, i1, i2, i3, ones_v, deg_l, norm_l,
                rows_a, rows_b,
                li0, li1, li2, li3, os0, os1, os2, os3,
                lsa, lsb, ssa, ssb):
        c = lax.axis_index("c")
        s = lax.axis_index("s")
        ebase = s * e_per_tile
        idxb = (i0, i1, i2, i3)
        lsem = (li0, li1, li2, li3)
        osem = (os0, os1, os2, os3)

        def i_start(k, p):
            @pl.when(c == 0)
            def _():
                pltpu.async_copy(
                    src_hbm.at[pl.ds(ebase + k * CH, CH)], idxb[p], lsem[p])

            @pl.when(c == 1)
            def _():
                pltpu.async_copy(
                    dst_hbm.at[pl.ds(ebase + k * CH, CH)], idxb[p], lsem[p])

        def i_wait(p):
            pltpu.make_async_copy(
                src_hbm.at[pl.ds(0, CH)], idxb[p], lsem[p]).wait()

        def a_start(k, p):
            pltpu.async_copy(ones_v, deg_sh.at[idxb[p]], osem[p], add=True)

        def a_wait(p):
            pltpu.make_async_copy(ones_v, deg_sh.at[idxb[p]], osem[p]).wait()

        for j in range(3):
            i_start(j, j)

        def ones_body(j, _):
            ones_v[pl.ds(j * L, L)] = jnp.full((L,), 1.0, jnp.float32)
            return 0
        lax.fori_loop(0, CH // L, ones_body, 0)
        _zero_vec(deg_l, per_node)
        pltpu.sync_copy(deg_l, deg_sh.at[pl.ds(s * per_node, per_node)])
        plsc.subcore_barrier()

        def dstep(k, p):
            p3 = (p + 3) % 4

            @pl.when(k + 3 < nch)
            def _():
                @pl.when(k >= 1)
                def _():
                    a_wait(p3)
                i_start(k + 3, p3)
            i_wait(p)
            a_start(k, p)

        def dpipe(kk, _):
            for b in range(4):
                dstep(kk * 4 + b, b)
            return 0
        lax.fori_loop(0, nch // 4, dpipe, 0)
        for t in range(nch - nch % 4, nch):
            dstep(t, t % 4)
        for t in range(nch - 4, nch):
            a_wait(t % 4)
        plsc.subcore_barrier()

        nbase = s * per_node
        pltpu.sync_copy(deg_sh.at[pl.ds(nbase, per_node)], deg_l)

        def norm_body(j, _):
            dv = jnp.maximum(deg_l[pl.ds(j * L, L)], jnp.float32(1.0))
            norm_l[pl.ds(j * L, L)] = _rsqrt16(dv)
            return 0
        lax.fori_loop(0, per_node // L, norm_body, 0)

        @pl.when(c == 1)
        def _():
            pltpu.sync_copy(norm_l, normdst_hbm.at[pl.ds(nbase, per_node)])

        @pl.when(c == 0)
        def _():
            nrows = jnp.minimum(per_node, jnp.maximum(n - nbase, 0))
            nrch = nrows // rch

            def load(q, buf, sem):
                pltpu.async_copy(
                    x_hbm.at[pl.ds(nbase + q * rch, rch)], buf, sem)

            def wait_load(buf, sem):
                pltpu.make_async_copy(x_hbm.at[pl.ds(0, rch)], buf, sem).wait()

            def store(q, buf, sem):
                pltpu.async_copy(
                    buf, h_hbm.at[pl.ds(nbase + q * rch, rch)], sem)

            def wait_store(buf, sem):
                pltpu.make_async_copy(buf, h_hbm.at[pl.ds(0, rch)], sem).wait()

            def scale(q, buf):
                def row_body(r, _):
                    sc = plsc.load_gather(
                        norm_l, [jnp.full((L,), q * rch + r, jnp.int32)])
                    for j in range(d // L):
                        buf[r, pl.ds(j * L, L)] = buf[r, pl.ds(j * L, L)] * sc
                    return 0
                lax.fori_loop(0, rch, row_body, 0)

            load(0, rows_a, lsa)

            def hpipe(qq, _):
                q = qq * 2
                @pl.when(q + 1 < nrch)
                def _():
                    @pl.when(q >= 1)
                    def _():
                        wait_store(rows_b, ssb)
                    load(q + 1, rows_b, lsb)
                wait_load(rows_a, lsa)
                scale(q, rows_a)
                store(q, rows_a, ssa)
                @pl.when(q + 1 < nrch)
                def _():
                    @pl.when(q + 2 < nrch)
                    def _():
                        wait_store(rows_a, ssa)
                        load(q + 2, rows_a, lsa)
                    wait_load(rows_b, lsb)
                    scale(q + 1, rows_b)
                    store(q + 1, rows_b, ssb)
                return 0
            lax.fori_loop(0, (nrch + 1) // 2, hpipe, 0)
            @pl.when(nrch >= 2)
            def _():
                wait_store(rows_b, ssb)
            wait_store(rows_a, ssa)

    return sc_prep


def _make_sc_agg(n, e, d):
    nw = NC * NS
    e_per_tile = e // nw
    nch = e_per_tile // CH
    rpt = (n // NS) // 8 * 8
    rpt_last = n - rpt * (NS - 1)
    assert e_per_tile % CH == 0 and nch >= 4

    mesh = plsc.VectorSubcoreMesh(
        core_axis_name="c", subcore_axis_name="s", num_cores=NC,
        num_subcores=NS)

    @functools.partial(
        pl.kernel,
        out_type=jax.ShapeDtypeStruct((NC, n, d), jnp.float32),
        mesh=mesh,
        scratch_types=[
            pltpu.VMEM_SHARED((n, d), jnp.float32),
            [pltpu.VMEM((CH,), jnp.int32)] * 8,
            [pltpu.VMEM((CH,), jnp.int32)] * 8,
            [pltpu.VMEM((CH, d), jnp.float32)] * 4,
            [pltpu.SemaphoreType.DMA] * 8,
            [pltpu.SemaphoreType.DMA] * 8,
            [pltpu.SemaphoreType.DMA] * 4,
            [pltpu.SemaphoreType.DMA] * 4,
        ],
        compiler_params=pltpu.CompilerParams(needs_layout_passes=False),
    )
    def sc_agg(h_hbm, src_hbm, dst_hbm, agg_hbm,
               agg_sh, sidx, didx, bufs, slsem, dlsem, gsem, ssem):
        c = lax.axis_index("c")
        s = lax.axis_index("s")
        wid = c * NS + s
        ebase = wid * e_per_tile

        def i_start(k, p):
            pltpu.async_copy(
                src_hbm.at[pl.ds(ebase + k * CH, CH)], sidx[p], slsem[p])
            pltpu.async_copy(
                dst_hbm.at[pl.ds(ebase + k * CH, CH)], didx[p], dlsem[p])

        def i_wait(p):
            pltpu.make_async_copy(
                src_hbm.at[pl.ds(0, CH)], sidx[p], slsem[p]).wait()
            pltpu.make_async_copy(
                src_hbm.at[pl.ds(0, CH)], didx[p], dlsem[p]).wait()

        def g_start(ip, p):
            pltpu.async_copy(h_hbm.at[sidx[ip]], bufs[p], gsem[p])

        def g_wait(ip, p):
            pltpu.make_async_copy(h_hbm.at[sidx[ip]], bufs[p], gsem[p]).wait()

        def s_start(ip, p):
            pltpu.async_copy(bufs[p], agg_sh.at[didx[ip]], ssem[p], add=True)

        def s_wait(ip, p):
            pltpu.make_async_copy(
                bufs[p], agg_sh.at[didx[ip]], ssem[p]).wait()

        for j in range(7):
            i_start(j, j)
        ba = bufs[0]
        _zero_rows(ba, CH, d)
        rbase = s * rpt

        @pl.when(s < NS - 1)
        def _():
            def zc(k, _):
                pltpu.sync_copy(ba, agg_sh.at[pl.ds(rbase + k * CH, CH)])
                return 0
            lax.fori_loop(0, rpt // CH, zc, 0)
            rem = rpt - (rpt // CH) * CH
            if rem:
                pltpu.sync_copy(
                    ba.at[pl.ds(0, rem)],
                    agg_sh.at[pl.ds(rbase + (rpt // CH) * CH, rem)])

        @pl.when(s == NS - 1)
        def _():
            def zc(k, _):
                pltpu.sync_copy(ba, agg_sh.at[pl.ds(rbase + k * CH, CH)])
                return 0
            lax.fori_loop(0, rpt_last // CH, zc, 0)
            rem = rpt_last - (rpt_last // CH) * CH
            if rem:
                pltpu.sync_copy(
                    ba.at[pl.ds(0, rem)],
                    agg_sh.at[pl.ds(rbase + (rpt_last // CH) * CH, rem)])

        plsc.subcore_barrier()

        for j in range(3):
            i_wait(j)
            g_start(j, j)

        def step1(k, b, when):
            p = b % 4
            p3 = (p + 3) % 4
            i3 = (b + 3) % 8
            i7 = (b + 7) % 8

            def prefetch():
                def free_ring():
                    s_wait(i7, p3)
                when(k >= 1, free_ring)

                def more_idx():
                    i_start(k + 7, i7)
                when(k + 7 < nch, more_idx)
                i_wait(i3)
                g_start(i3, p3)
            when(k + 3 < nch, prefetch)
            g_wait(b, p)
            s_start(b, p)

        def twhen(cond, fn):
            pl.when(cond)(fn)

        def pipe1(kk, _):
            for b in range(8):
                step1(kk * 8 + b, b, twhen)
            return 0
        lax.fori_loop(0, nch // 8, pipe1, 0)

        def swhen(cond, fn):
            if cond:
                fn()
        base = nch - nch % 8
        for t in range(base, nch):
            step1(t, t % 8, swhen)
        for t in range(nch - 4, nch):
            s_wait(t % 8, t % 4)
        plsc.subcore_barrier()

        @pl.when(s < NS - 1)
        def _():
            pltpu.sync_copy(agg_sh.at[pl.ds(rbase, rpt)],
                            agg_hbm.at[c, pl.ds(rbase, rpt)])

        @pl.when(s == NS - 1)
        def _():
            pltpu.sync_copy(agg_sh.at[pl.ds(rbase, rpt_last)],
                            agg_hbm.at[c, pl.ds(rbase, rpt_last)])

    return sc_agg


def _tc_final_body(agg_ref, nd_ref, w_ref, b_ref, o_ref):
    a = (agg_ref[0] + agg_ref[1]) * nd_ref[...]
    w = jnp.maximum(w_ref[...], 0.0)
    r = jnp.dot(a, w, preferred_element_type=jnp.float32) + b_ref[...]
    o_ref[...] = jnp.where(r >= 0, r, jnp.float32(0.01) * r)


def _make_tc_final(n, d, blk):
    grid = n // blk
    return pl.pallas_call(
        _tc_final_body,
        grid=(grid,),
        in_specs=[
            pl.BlockSpec((NC, blk, d), lambda i: (0, i, 0)),
            pl.BlockSpec((blk, 1), lambda i: (i, 0)),
            pl.BlockSpec((d, d), lambda i: (0, 0)),
            pl.BlockSpec((1, d), lambda i: (0, 0)),
        ],
        out_specs=pl.BlockSpec((blk, d), lambda i: (i, 0)),
        out_shape=jax.ShapeDtypeStruct((n, d), jnp.float32),
    )


@jax.jit
def kernel(x, edge_index, W, b):
    n, d = x.shape
    e = edge_index.shape[1]
    npad = ((n + NC * NS * L - 1) // (NC * NS * L)) * NC * NS * L

    src = edge_index[0]
    dst = edge_index[1]

    h, norm_dst = _make_sc_prep(n, npad, e, d)(x, src, dst)
    aggp = _make_sc_agg(n, e, d)(h, src, dst)
    out = _make_tc_final(n, d, 1000)(
        aggp, norm_dst[:n, None], W, b[None, :])
    return out


# speedup vs baseline: 13.5108x; 1.0644x over previous
import functools

import jax
import jax.numpy as jnp
from jax import lax
from jax.experimental import pallas as pl
from jax.experimental.pallas import tpu as pltpu
from jax.experimental.pallas import tpu_sc as plsc

NC = 2
NS = 16
L = 16
CH = 80


def _rsqrt16(d):
    i = lax.bitcast_convert_type(d, jnp.int32)
    i = jnp.int32(0x5F3759DF) - (i >> 1)
    y = lax.bitcast_convert_type(i, jnp.float32)
    for _ in range(3):
        y = y * (jnp.float32(1.5) - jnp.float32(0.5) * d * y * y)
    return y


def _zero_vec(ref, n):
    def body(j, _):
        ref[pl.ds(j * L, L)] = jnp.zeros((L,), jnp.float32)
        return 0
    lax.fori_loop(0, n // L, body, 0)


def _zero_rows(ref, rows, cols):
    def rbody(r, _):
        def cbody(j, _):
            ref[r, pl.ds(j * L, L)] = jnp.zeros((L,), jnp.float32)
            return 0
        lax.fori_loop(0, cols // L, cbody, 0)
        return 0
    lax.fori_loop(0, rows, rbody, 0)


def _make_sc_prep(n, npad, e, d):
    per_node = npad // NS
    e_per_tile = e // NS
    ECH = 128
    nch = e_per_tile // ECH
    tail = e_per_tile - nch * ECH
    rch = 80
    assert per_node % rch == 0 and n % rch == 0 and tail % 8 == 0

    mesh = plsc.VectorSubcoreMesh(
        core_axis_name="c", subcore_axis_name="s", num_cores=NC,
        num_subcores=NS)

    @functools.partial(
        pl.kernel,
        out_type=[
            jax.ShapeDtypeStruct((n, d), jnp.float32),
            jax.ShapeDtypeStruct((npad,), jnp.float32),
        ],
        mesh=mesh,
        scratch_types=[
            pltpu.VMEM_SHARED((npad,), jnp.float32),
            [pltpu.VMEM((ECH,), jnp.int32)] * 4,
            pltpu.VMEM((tail,), jnp.int32),
            pltpu.VMEM((ECH,), jnp.float32),
            pltpu.VMEM((per_node,), jnp.float32),
            pltpu.VMEM((per_node,), jnp.float32),
            pltpu.VMEM((rch, d), jnp.float32),
            pltpu.VMEM((rch, d), jnp.float32),
            [pltpu.SemaphoreType.DMA] * 4,
            [pltpu.SemaphoreType.DMA] * 4,
            pltpu.SemaphoreType.DMA,
            pltpu.SemaphoreType.DMA,
            pltpu.SemaphoreType.DMA,
            pltpu.SemaphoreType.DMA,
        ],
        compiler_params=pltpu.CompilerParams(needs_layout_passes=False),
    )
    def sc_prep(x_hbm, src_hbm, dst_hbm, h_hbm, normdst_hbm,
                deg_sh, idxb, idxt, ones_v, deg_l, norm_l,
                rows_a, rows_b,
                lsem, osem, lsa, lsb, ssa, ssb):
        c = lax.axis_index("c")
        s = lax.axis_index("s")
        ebase = s * e_per_tile

        def e_slice(lo, sz, dstref, sem):
            @pl.when(c == 0)
            def _():
                pltpu.async_copy(src_hbm.at[pl.ds(lo, sz)], dstref, sem)

            @pl.when(c == 1)
            def _():
                pltpu.async_copy(dst_hbm.at[pl.ds(lo, sz)], dstref, sem)

        def i_start(k, p):
            e_slice(ebase + k * ECH, ECH, idxb[p], lsem[p])

        def i_wait(p):
            pltpu.make_async_copy(
                src_hbm.at[pl.ds(0, ECH)], idxb[p], lsem[p]).wait()

        def a_start(k, p):
            pltpu.async_copy(ones_v, deg_sh.at[idxb[p]], osem[p], add=True)

        def a_wait(p):
            pltpu.make_async_copy(ones_v, deg_sh.at[idxb[p]], osem[p]).wait()

        for j in range(3):
            i_start(j, j)

        def ones_body(j, _):
            ones_v[pl.ds(j * L, L)] = jnp.full((L,), 1.0, jnp.float32)
            return 0
        lax.fori_loop(0, ECH // L, ones_body, 0)
        _zero_vec(deg_l, per_node)
        pltpu.sync_copy(deg_l, deg_sh.at[pl.ds(s * per_node, per_node)])
        plsc.subcore_barrier()

        def dstep(k, p):
            p3 = (p + 3) % 4

            @pl.when(k + 3 < nch)
            def _():
                @pl.when(k >= 1)
                def _():
                    a_wait(p3)
                i_start(k + 3, p3)
            i_wait(p)
            a_start(k, p)

        def dpipe(kk, _):
            for b in range(4):
                dstep(kk * 4 + b, b)
            return 0
        lax.fori_loop(0, nch // 4, dpipe, 0)
        for t in range(nch - nch % 4, nch):
            dstep(t, t % 4)
        for t in range(nch - 4, nch):
            a_wait(t % 4)
        if tail:
            e_slice(ebase + nch * ECH, tail, idxt, lsem[0])
            pltpu.make_async_copy(
                src_hbm.at[pl.ds(0, tail)], idxt, lsem[0]).wait()
            pltpu.sync_copy(ones_v.at[pl.ds(0, tail)],
                            deg_sh.at[idxt], add=True)
        plsc.subcore_barrier()

        nbase = s * per_node
        pltpu.sync_copy(deg_sh.at[pl.ds(nbase, per_node)], deg_l)

        def norm_body(j, _):
            dv = jnp.maximum(deg_l[pl.ds(j * L, L)], jnp.float32(1.0))
            norm_l[pl.ds(j * L, L)] = _rsqrt16(dv)
            return 0
        lax.fori_loop(0, per_node // L, norm_body, 0)

        @pl.when(c == 1)
        def _():
            pltpu.sync_copy(norm_l, normdst_hbm.at[pl.ds(nbase, per_node)])

        @pl.when(c == 0)
        def _():
            nrows = jnp.minimum(per_node, jnp.maximum(n - nbase, 0))
            nrch = nrows // rch

            def load(q, buf, sem):
                pltpu.async_copy(
                    x_hbm.at[pl.ds(nbase + q * rch, rch)], buf, sem)

            def wait_load(buf, sem):
                pltpu.make_async_copy(x_hbm.at[pl.ds(0, rch)], buf, sem).wait()

            def store(q, buf, sem):
                pltpu.async_copy(
                    buf, h_hbm.at[pl.ds(nbase + q * rch, rch)], sem)

            def wait_store(buf, sem):
                pltpu.make_async_copy(buf, h_hbm.at[pl.ds(0, rch)], sem).wait()

            def scale(q, buf):
                def row_body(rr, _):
                    for u in range(2):
                        r = rr * 2 + u
                        sc = plsc.load_gather(
                            norm_l, [jnp.full((L,), q * rch + r, jnp.int32)])
                        for j in range(d // L):
                            buf[r, pl.ds(j * L, L)] = (
                                buf[r, pl.ds(j * L, L)] * sc)
                    return 0
                lax.fori_loop(0, rch // 2, row_body, 0)

            load(0, rows_a, lsa)

            def hpipe(qq, _):
                q = qq * 2
                @pl.when(q + 1 < nrch)
                def _():
                    @pl.when(q >= 1)
                    def _():
                        wait_store(rows_b, ssb)
                    load(q + 1, rows_b, lsb)
                wait_load(rows_a, lsa)
                scale(q, rows_a)
                store(q, rows_a, ssa)
                @pl.when(q + 1 < nrch)
                def _():
                    @pl.when(q + 2 < nrch)
                    def _():
                        wait_store(rows_a, ssa)
                        load(q + 2, rows_a, lsa)
                    wait_load(rows_b, lsb)
                    scale(q + 1, rows_b)
                    store(q + 1, rows_b, ssb)
                return 0
            lax.fori_loop(0, (nrch + 1) // 2, hpipe, 0)
            @pl.when(nrch >= 2)
            def _():
                wait_store(rows_b, ssb)
            wait_store(rows_a, ssa)

    return sc_prep


def _make_sc_agg(n, e, d):
    nw = NC * NS
    e_per_tile = e // nw
    nch = e_per_tile // CH
    rpt = (n // NS) // 8 * 8
    rpt_last = n - rpt * (NS - 1)
    assert e_per_tile % CH == 0 and nch >= 4

    mesh = plsc.VectorSubcoreMesh(
        core_axis_name="c", subcore_axis_name="s", num_cores=NC,
        num_subcores=NS)

    @functools.partial(
        pl.kernel,
        out_type=jax.ShapeDtypeStruct((NC, n, d), jnp.float32),
        mesh=mesh,
        scratch_types=[
            pltpu.VMEM_SHARED((n, d), jnp.float32),
            [pltpu.VMEM((CH,), jnp.int32)] * 8,
            [pltpu.VMEM((CH,), jnp.int32)] * 8,
            [pltpu.VMEM((CH, d), jnp.float32)] * 4,
            [pltpu.SemaphoreType.DMA] * 8,
            [pltpu.SemaphoreType.DMA] * 8,
            [pltpu.SemaphoreType.DMA] * 4,
            [pltpu.SemaphoreType.DMA] * 4,
        ],
        compiler_params=pltpu.CompilerParams(needs_layout_passes=False),
    )
    def sc_agg(h_hbm, src_hbm, dst_hbm, agg_hbm,
               agg_sh, sidx, didx, bufs, slsem, dlsem, gsem, ssem):
        c = lax.axis_index("c")
        s = lax.axis_index("s")
        wid = c * NS + s
        ebase = wid * e_per_tile

        def i_start(k, p):
            pltpu.async_copy(
                src_hbm.at[pl.ds(ebase + k * CH, CH)], sidx[p], slsem[p])
            pltpu.async_copy(
                dst_hbm.at[pl.ds(ebase + k * CH, CH)], didx[p], dlsem[p])

        def i_wait(p):
            pltpu.make_async_copy(
                src_hbm.at[pl.ds(0, CH)], sidx[p], slsem[p]).wait()
            pltpu.make_async_copy(
                src_hbm.at[pl.ds(0, CH)], didx[p], dlsem[p]).wait()

        def g_start(ip, p):
            pltpu.async_copy(h_hbm.at[sidx[ip]], bufs[p], gsem[p])

        def g_wait(ip, p):
            pltpu.make_async_copy(h_hbm.at[sidx[ip]], bufs[p], gsem[p]).wait()

        def s_start(ip, p):
            pltpu.async_copy(bufs[p], agg_sh.at[didx[ip]], ssem[p], add=True)

        def s_wait(ip, p):
            pltpu.make_async_copy(
                bufs[p], agg_sh.at[didx[ip]], ssem[p]).wait()

        for j in range(7):
            i_start(j, j)
        ba = bufs[0]
        _zero_rows(ba, CH, d)
        rbase = s * rpt

        @pl.when(s < NS - 1)
        def _():
            def zc(k, _):
                pltpu.sync_copy(ba, agg_sh.at[pl.ds(rbase + k * CH, CH)])
                return 0
            lax.fori_loop(0, rpt // CH, zc, 0)
            rem = rpt - (rpt // CH) * CH
            if rem:
                pltpu.sync_copy(
                    ba.at[pl.ds(0, rem)],
                    agg_sh.at[pl.ds(rbase + (rpt // CH) * CH, rem)])

        @pl.when(s == NS - 1)
        def _():
            def zc(k, _):
                pltpu.sync_copy(ba, agg_sh.at[pl.ds(rbase + k * CH, CH)])
                return 0
            lax.fori_loop(0, rpt_last // CH, zc, 0)
            rem = rpt_last - (rpt_last // CH) * CH
            if rem:
                pltpu.sync_copy(
                    ba.at[pl.ds(0, rem)],
                    agg_sh.at[pl.ds(rbase + (rpt_last // CH) * CH, rem)])

        plsc.subcore_barrier()

        for j in range(3):
            i_wait(j)
            g_start(j, j)

        def step1(k, b, when):
            p = b % 4
            p3 = (p + 3) % 4
            i3 = (b + 3) % 8
            i7 = (b + 7) % 8

            def prefetch():
                def free_ring():
                    s_wait(i7, p3)
                when(k >= 1, free_ring)

                def more_idx():
                    i_start(k + 7, i7)
                when(k + 7 < nch, more_idx)
                i_wait(i3)
                g_start(i3, p3)
            when(k + 3 < nch, prefetch)
            g_wait(b, p)
            s_start(b, p)

        def twhen(cond, fn):
            pl.when(cond)(fn)

        def pipe1(kk, _):
            for b in range(8):
                step1(kk * 8 + b, b, twhen)
            return 0
        lax.fori_loop(0, nch // 8, pipe1, 0)

        def swhen(cond, fn):
            if cond:
                fn()
        base = nch - nch % 8
        for t in range(base, nch):
            step1(t, t % 8, swhen)
        for t in range(nch - 4, nch):
            s_wait(t % 8, t % 4)
        plsc.subcore_barrier()

        @pl.when(s < NS - 1)
        def _():
            pltpu.sync_copy(agg_sh.at[pl.ds(rbase, rpt)],
                            agg_hbm.at[c, pl.ds(rbase, rpt)])

        @pl.when(s == NS - 1)
        def _():
            pltpu.sync_copy(agg_sh.at[pl.ds(rbase, rpt_last)],
                            agg_hbm.at[c, pl.ds(rbase, rpt_last)])

    return sc_agg


def _tc_final_body(agg_ref, nd_ref, w_ref, b_ref, o_ref):
    a = (agg_ref[0] + agg_ref[1]) * nd_ref[...]
    w = jnp.maximum(w_ref[...], 0.0)
    r = jnp.dot(a, w, preferred_element_type=jnp.float32) + b_ref[...]
    o_ref[...] = jnp.where(r >= 0, r, jnp.float32(0.01) * r)


def _make_tc_final(n, d, blk):
    grid = n // blk
    return pl.pallas_call(
        _tc_final_body,
        grid=(grid,),
        in_specs=[
            pl.BlockSpec((NC, blk, d), lambda i: (0, i, 0)),
            pl.BlockSpec((blk, 1), lambda i: (i, 0)),
            pl.BlockSpec((d, d), lambda i: (0, 0)),
            pl.BlockSpec((1, d), lambda i: (0, 0)),
        ],
        out_specs=pl.BlockSpec((blk, d), lambda i: (i, 0)),
        out_shape=jax.ShapeDtypeStruct((n, d), jnp.float32),
    )


@jax.jit
def kernel(x, edge_index, W, b):
    n, d = x.shape
    e = edge_index.shape[1]
    npad = ((n + NC * NS * L - 1) // (NC * NS * L)) * NC * NS * L

    src = edge_index[0]
    dst = edge_index[1]

    h, norm_dst = _make_sc_prep(n, npad, e, d)(x, src, dst)
    aggp = _make_sc_agg(n, e, d)(h, src, dst)
    out = _make_tc_final(n, d, 1000)(
        aggp, norm_dst[:n, None], W, b[None, :])
    return out


# speedup vs baseline: 13.5947x; 1.0062x over previous
import functools

import jax
import jax.numpy as jnp
from jax import lax
from jax.experimental import pallas as pl
from jax.experimental.pallas import tpu as pltpu
from jax.experimental.pallas import tpu_sc as plsc

NC = 2
NS = 16
L = 16
CH = 80


def _rsqrt16(d):
    i = lax.bitcast_convert_type(d, jnp.int32)
    i = jnp.int32(0x5F3759DF) - (i >> 1)
    y = lax.bitcast_convert_type(i, jnp.float32)
    for _ in range(3):
        y = y * (jnp.float32(1.5) - jnp.float32(0.5) * d * y * y)
    return y


def _zero_vec(ref, n):
    def body(j, _):
        ref[pl.ds(j * L, L)] = jnp.zeros((L,), jnp.float32)
        return 0
    lax.fori_loop(0, n // L, body, 0)


def _zero_rows(ref, rows, cols):
    def rbody(r, _):
        def cbody(j, _):
            ref[r, pl.ds(j * L, L)] = jnp.zeros((L,), jnp.float32)
            return 0
        lax.fori_loop(0, cols // L, cbody, 0)
        return 0
    lax.fori_loop(0, rows, rbody, 0)


def _make_sc_prep(n, npad, e, d):
    per_node = npad // NS
    e_per_tile = e // NS
    ECH = 128
    nch = e_per_tile // ECH
    tail = e_per_tile - nch * ECH
    rch = 80
    assert per_node % rch == 0 and n % rch == 0 and tail % 8 == 0

    mesh = plsc.VectorSubcoreMesh(
        core_axis_name="c", subcore_axis_name="s", num_cores=NC,
        num_subcores=NS)

    @functools.partial(
        pl.kernel,
        out_type=[
            jax.ShapeDtypeStruct((n, d), jnp.float32),
            jax.ShapeDtypeStruct((npad,), jnp.float32),
        ],
        mesh=mesh,
        scratch_types=[
            pltpu.VMEM_SHARED((npad,), jnp.float32),
            [pltpu.VMEM((ECH,), jnp.int32)] * 4,
            pltpu.VMEM((tail,), jnp.int32),
            pltpu.VMEM((ECH,), jnp.float32),
            pltpu.VMEM((per_node,), jnp.float32),
            pltpu.VMEM((per_node,), jnp.float32),
            pltpu.VMEM((rch, d), jnp.float32),
            pltpu.VMEM((rch, d), jnp.float32),
            [pltpu.SemaphoreType.DMA] * 4,
            [pltpu.SemaphoreType.DMA] * 4,
            pltpu.SemaphoreType.DMA,
            pltpu.SemaphoreType.DMA,
            pltpu.SemaphoreType.DMA,
            pltpu.SemaphoreType.DMA,
        ],
        compiler_params=pltpu.CompilerParams(needs_layout_passes=False),
    )
    def sc_prep(x_hbm, src_hbm, dst_hbm, h_hbm, normdst_hbm,
                deg_sh, idxb, idxt, ones_v, deg_l, norm_l,
                rows_a, rows_b,
                lsem, osem, lsa, lsb, ssa, ssb):
        c = lax.axis_index("c")
        s = lax.axis_index("s")
        ebase = s * e_per_tile

        def e_slice(lo, sz, dstref, sem):
            @pl.when(c == 0)
            def _():
                pltpu.async_copy(src_hbm.at[pl.ds(lo, sz)], dstref, sem)

            @pl.when(c == 1)
            def _():
                pltpu.async_copy(dst_hbm.at[pl.ds(lo, sz)], dstref, sem)

        def i_start(k, p):
            e_slice(ebase + k * ECH, ECH, idxb[p], lsem[p])

        def i_wait(p):
            pltpu.make_async_copy(
                src_hbm.at[pl.ds(0, ECH)], idxb[p], lsem[p]).wait()

        def a_start(k, p):
            pltpu.async_copy(ones_v, deg_sh.at[idxb[p]], osem[p], add=True)

        def a_wait(p):
            pltpu.make_async_copy(ones_v, deg_sh.at[idxb[p]], osem[p]).wait()

        for j in range(3):
            i_start(j, j)

        def ones_body(j, _):
            ones_v[pl.ds(j * L, L)] = jnp.full((L,), 1.0, jnp.float32)
            return 0
        lax.fori_loop(0, ECH // L, ones_body, 0)
        _zero_vec(deg_l, per_node)
        pltpu.sync_copy(deg_l, deg_sh.at[pl.ds(s * per_node, per_node)])
        plsc.subcore_barrier()

        def dstep(k, p):
            p3 = (p + 3) % 4

            @pl.when(k + 3 < nch)
            def _():
                @pl.when(k >= 1)
                def _():
                    a_wait(p3)
                i_start(k + 3, p3)
            i_wait(p)
            a_start(k, p)

        def dpipe(kk, _):
            for b in range(4):
                dstep(kk * 4 + b, b)
            return 0
        lax.fori_loop(0, nch // 4, dpipe, 0)
        for t in range(nch - nch % 4, nch):
            dstep(t, t % 4)
        for t in range(nch - 4, nch):
            a_wait(t % 4)
        if tail:
            e_slice(ebase + nch * ECH, tail, idxt, lsem[0])
            pltpu.make_async_copy(
                src_hbm.at[pl.ds(0, tail)], idxt, lsem[0]).wait()
            pltpu.sync_copy(ones_v.at[pl.ds(0, tail)],
                            deg_sh.at[idxt], add=True)
        plsc.subcore_barrier()

        nbase = s * per_node
        pltpu.sync_copy(deg_sh.at[pl.ds(nbase, per_node)], deg_l)

        def norm_body(j, _):
            dv = jnp.maximum(deg_l[pl.ds(j * L, L)], jnp.float32(1.0))
            norm_l[pl.ds(j * L, L)] = _rsqrt16(dv)
            return 0
        lax.fori_loop(0, per_node // L, norm_body, 0)

        @pl.when(c == 1)
        def _():
            pltpu.sync_copy(norm_l, normdst_hbm.at[pl.ds(nbase, per_node)])

        @pl.when(c == 0)
        def _():
            nrows = jnp.minimum(per_node, jnp.maximum(n - nbase, 0))
            nrch = nrows // rch

            def load(q, buf, sem):
                pltpu.async_copy(
                    x_hbm.at[pl.ds(nbase + q * rch, rch)], buf, sem)

            def wait_load(buf, sem):
                pltpu.make_async_copy(x_hbm.at[pl.ds(0, rch)], buf, sem).wait()

            def store(q, buf, sem):
                pltpu.async_copy(
                    buf, h_hbm.at[pl.ds(nbase + q * rch, rch)], sem)

            def wait_store(buf, sem):
                pltpu.make_async_copy(buf, h_hbm.at[pl.ds(0, rch)], sem).wait()

            def scale(q, buf):
                def row_body(rr, _):
                    for u in range(2):
                        r = rr * 2 + u
                        sc = plsc.load_gather(
                            norm_l, [jnp.full((L,), q * rch + r, jnp.int32)])
                        for j in range(d // L):
                            buf[r, pl.ds(j * L, L)] = (
                                buf[r, pl.ds(j * L, L)] * sc)
                    return 0
                lax.fori_loop(0, rch // 2, row_body, 0)

            load(0, rows_a, lsa)

            def hpipe(qq, _):
                q = qq * 2
                @pl.when(q + 1 < nrch)
                def _():
                    @pl.when(q >= 1)
                    def _():
                        wait_store(rows_b, ssb)
                    load(q + 1, rows_b, lsb)
                wait_load(rows_a, lsa)
                scale(q, rows_a)
                store(q, rows_a, ssa)
                @pl.when(q + 1 < nrch)
                def _():
                    @pl.when(q + 2 < nrch)
                    def _():
                        wait_store(rows_a, ssa)
                        load(q + 2, rows_a, lsa)
                    wait_load(rows_b, lsb)
                    scale(q + 1, rows_b)
                    store(q + 1, rows_b, ssb)
                return 0
            lax.fori_loop(0, (nrch + 1) // 2, hpipe, 0)
            @pl.when(nrch >= 2)
            def _():
                wait_store(rows_b, ssb)
            wait_store(rows_a, ssa)

    return sc_prep


def _make_sc_agg(n, e, d):
    nw = NC * NS
    e_per_tile = e // nw
    nch = e_per_tile // CH
    rpt = (n // NS) // 8 * 8
    rpt_last = n - rpt * (NS - 1)
    assert e_per_tile % CH == 0 and nch >= 4

    mesh = plsc.VectorSubcoreMesh(
        core_axis_name="c", subcore_axis_name="s", num_cores=NC,
        num_subcores=NS)

    @functools.partial(
        pl.kernel,
        out_type=jax.ShapeDtypeStruct((NC, n, d), jnp.float32),
        mesh=mesh,
        scratch_types=[
            pltpu.VMEM_SHARED((n, d), jnp.float32),
            [pltpu.VMEM((CH,), jnp.int32)] * 8,
            [pltpu.VMEM((CH,), jnp.int32)] * 8,
            [pltpu.VMEM((CH, d), jnp.float32)] * 4,
            [pltpu.SemaphoreType.DMA] * 8,
            [pltpu.SemaphoreType.DMA] * 8,
            [pltpu.SemaphoreType.DMA] * 4,
            [pltpu.SemaphoreType.DMA] * 4,
        ],
        compiler_params=pltpu.CompilerParams(needs_layout_passes=False),
    )
    def sc_agg(h_hbm, src_hbm, dst_hbm, agg_hbm,
               agg_sh, sidx, didx, bufs, slsem, dlsem, gsem, ssem):
        c = lax.axis_index("c")
        s = lax.axis_index("s")
        wid = c * NS + s
        ebase = wid * e_per_tile

        def i_start(k, p):
            pltpu.async_copy(
                src_hbm.at[pl.ds(ebase + k * CH, CH)], sidx[p], slsem[p])
            pltpu.async_copy(
                dst_hbm.at[pl.ds(ebase + k * CH, CH)], didx[p], dlsem[p])

        def i_wait(p):
            pltpu.make_async_copy(
                src_hbm.at[pl.ds(0, CH)], sidx[p], slsem[p]).wait()
            pltpu.make_async_copy(
                src_hbm.at[pl.ds(0, CH)], didx[p], dlsem[p]).wait()

        def g_start(ip, p):
            pltpu.async_copy(h_hbm.at[sidx[ip]], bufs[p], gsem[p])

        def g_wait(ip, p):
            pltpu.make_async_copy(h_hbm.at[sidx[ip]], bufs[p], gsem[p]).wait()

        def s_start(ip, p):
            pltpu.async_copy(bufs[p], agg_sh.at[didx[ip]], ssem[p], add=True)

        def s_wait(ip, p):
            pltpu.make_async_copy(
                bufs[p], agg_sh.at[didx[ip]], ssem[p]).wait()

        for j in range(7):
            i_start(j, j)
        for j in range(3):
            i_wait(j)
            g_start(j, j)
        ba = bufs[3]
        _zero_rows(ba, CH, d)
        rbase = s * rpt

        @pl.when(s < NS - 1)
        def _():
            def zc(k, _):
                pltpu.sync_copy(ba, agg_sh.at[pl.ds(rbase + k * CH, CH)])
                return 0
            lax.fori_loop(0, rpt // CH, zc, 0)
            rem = rpt - (rpt // CH) * CH
            if rem:
                pltpu.sync_copy(
                    ba.at[pl.ds(0, rem)],
                    agg_sh.at[pl.ds(rbase + (rpt // CH) * CH, rem)])

        @pl.when(s == NS - 1)
        def _():
            def zc(k, _):
                pltpu.sync_copy(ba, agg_sh.at[pl.ds(rbase + k * CH, CH)])
                return 0
            lax.fori_loop(0, rpt_last // CH, zc, 0)
            rem = rpt_last - (rpt_last // CH) * CH
            if rem:
                pltpu.sync_copy(
                    ba.at[pl.ds(0, rem)],
                    agg_sh.at[pl.ds(rbase + (rpt_last // CH) * CH, rem)])

        plsc.subcore_barrier()

        def step1(k, b, when):
            p = b % 4
            p3 = (p + 3) % 4
            i3 = (b + 3) % 8
            i7 = (b + 7) % 8

            def prefetch():
                def free_ring():
                    s_wait(i7, p3)
                when(k >= 1, free_ring)

                def more_idx():
                    i_start(k + 7, i7)
                when(k + 7 < nch, more_idx)
                i_wait(i3)
                g_start(i3, p3)
            when(k + 3 < nch, prefetch)
            g_wait(b, p)
            s_start(b, p)

        def twhen(cond, fn):
            pl.when(cond)(fn)

        def pipe1(kk, _):
            for b in range(8):
                step1(kk * 8 + b, b, twhen)
            return 0
        lax.fori_loop(0, nch // 8, pipe1, 0)

        def swhen(cond, fn):
            if cond:
                fn()
        base = nch - nch % 8
        for t in range(base, nch):
            step1(t, t % 8, swhen)
        for t in range(nch - 4, nch):
            s_wait(t % 8, t % 4)
        plsc.subcore_barrier()

        @pl.when(s < NS - 1)
        def _():
            pltpu.sync_copy(agg_sh.at[pl.ds(rbase, rpt)],
                            agg_hbm.at[c, pl.ds(rbase, rpt)])

        @pl.when(s == NS - 1)
        def _():
            pltpu.sync_copy(agg_sh.at[pl.ds(rbase, rpt_last)],
                            agg_hbm.at[c, pl.ds(rbase, rpt_last)])

    return sc_agg


def _tc_final_body(agg_ref, nd_ref, w_ref, b_ref, o_ref):
    a = (agg_ref[0] + agg_ref[1]) * nd_ref[...]
    w = jnp.maximum(w_ref[...], 0.0)
    r = jnp.dot(a, w, preferred_element_type=jnp.float32) + b_ref[...]
    o_ref[...] = jnp.where(r >= 0, r, jnp.float32(0.01) * r)


def _make_tc_final(n, d, blk):
    grid = n // blk
    return pl.pallas_call(
        _tc_final_body,
        grid=(grid,),
        in_specs=[
            pl.BlockSpec((NC, blk, d), lambda i: (0, i, 0)),
            pl.BlockSpec((blk, 1), lambda i: (i, 0)),
            pl.BlockSpec((d, d), lambda i: (0, 0)),
            pl.BlockSpec((1, d), lambda i: (0, 0)),
        ],
        out_specs=pl.BlockSpec((blk, d), lambda i: (i, 0)),
        out_shape=jax.ShapeDtypeStruct((n, d), jnp.float32),
    )


@jax.jit
def kernel(x, edge_index, W, b):
    n, d = x.shape
    e = edge_index.shape[1]
    npad = ((n + NC * NS * L - 1) // (NC * NS * L)) * NC * NS * L

    src = edge_index[0]
    dst = edge_index[1]

    h, norm_dst = _make_sc_prep(n, npad, e, d)(x, src, dst)
    aggp = _make_sc_agg(n, e, d)(h, src, dst)
    out = _make_tc_final(n, d, 1000)(
        aggp, norm_dst[:n, None], W, b[None, :])
    return out
